# baseline (speedup 1.0000x reference)
import functools

import jax
import jax.numpy as jnp
from jax import lax
from jax.experimental import pallas as pl
from jax.experimental.pallas import tpu as pltpu, tpu_sc as plsc

N_CLASSES = 4
C_Z = 128
EPS = 1e-5

NC, NS = 2, 16
NW = NC * NS
CHUNK = 128


def _table_block(embed_ref, gamma_ref, beta_ref, w_ref, out_ref):
    e = embed_ref[:, :]
    mean = jnp.mean(e, axis=1, keepdims=True)
    var = jnp.mean(jnp.square(e - mean), axis=1, keepdims=True)
    norm = (e - mean) * lax.rsqrt(var + EPS) * gamma_ref[:, :] + beta_ref[:, :]
    out_ref[:, :] = lax.dot_general(norm, w_ref[:, :], (((1,), (1,)), ((), ())),
                                    preferred_element_type=jnp.float32)


def _make_sc_expand(total):
    rows_per_w = total // NW
    n_chunks = rows_per_w // CHUNK
    mesh = plsc.VectorSubcoreMesh(core_axis_name="c", subcore_axis_name="s")

    @functools.partial(
        pl.kernel,
        mesh=mesh,
        out_type=jax.ShapeDtypeStruct((total, C_Z), jnp.float32),
        scratch_types=[
            pltpu.VMEM((CHUNK,), jnp.int32),
            pltpu.VMEM((CHUNK, C_Z), jnp.float32),
            pltpu.SemaphoreType.DMA,
        ],
    )
    def sc_expand(table_hbm, ids_hbm, out_hbm, idx_v, rows_v, sem):
        wid = lax.axis_index("s") * NC + lax.axis_index("c")
        base = wid * rows_per_w

        def body(g, carry):
            off = base + g * CHUNK
            pltpu.sync_copy(ids_hbm.at[pl.ds(off, CHUNK)], idx_v)
            pltpu.async_copy(table_hbm.at[idx_v], rows_v, sem).wait()
            pltpu.sync_copy(rows_v, out_hbm.at[pl.ds(off, CHUNK)])
            return carry

        lax.fori_loop(0, n_chunks, body, 0)

    return sc_expand


@jax.jit
def kernel(ss_matrix, embed_table, ln_gamma, ln_beta, W):
    b, n, _ = ss_matrix.shape
    total = b * n * n
    table4 = pl.pallas_call(
        _table_block,
        out_shape=jax.ShapeDtypeStruct((N_CLASSES, C_Z), jnp.float32),
    )(embed_table, ln_gamma.reshape(1, C_Z), ln_beta.reshape(1, C_Z), W)
    ids = ss_matrix.reshape(total)
    out = _make_sc_expand(total)(table4, ids)
    return out.reshape(b, n, n, C_Z)


# speedup vs baseline: 1.0004x; 1.0004x over previous
import functools

import jax
import jax.numpy as jnp
from jax import lax
from jax.experimental import pallas as pl
from jax.experimental.pallas import tpu as pltpu, tpu_sc as plsc

N_CLASSES = 4
C_Z = 128
EPS = 1e-5

NC, NS = 2, 16
NW = NC * NS
CHUNK = 128
NBUF = 4


def _table_block(embed_ref, gamma_ref, beta_ref, w_ref, out_ref):
    e = embed_ref[:, :]
    mean = jnp.mean(e, axis=1, keepdims=True)
    var = jnp.mean(jnp.square(e - mean), axis=1, keepdims=True)
    norm = (e - mean) * lax.rsqrt(var + EPS) * gamma_ref[:, :] + beta_ref[:, :]
    out_ref[:, :] = lax.dot_general(norm, w_ref[:, :], (((1,), (1,)), ((), ())),
                                    preferred_element_type=jnp.float32)


def _make_sc_expand(total):
    rows_per_w = total // NW
    n_chunks = rows_per_w // CHUNK
    n_super = n_chunks // NBUF
    mesh = plsc.VectorSubcoreMesh(core_axis_name="c", subcore_axis_name="s")

    @functools.partial(
        pl.kernel,
        mesh=mesh,
        out_type=jax.ShapeDtypeStruct((total, C_Z), jnp.float32),
        scratch_types=[
            pltpu.VMEM((n_chunks, CHUNK), jnp.int32),
            pltpu.VMEM((NBUF, CHUNK, C_Z), jnp.float32),
            pltpu.SemaphoreType.DMA,
            pltpu.SemaphoreType.DMA,
        ],
    )
    def sc_expand(table_hbm, ids_hbm, out_hbm, idx_all, bufs, gsem, ssem):
        wid = lax.axis_index("s") * NC + lax.axis_index("c")
        base = wid * rows_per_w
        pltpu.sync_copy(ids_hbm.at[wid], idx_all)

        def body(sc, carry):
            off0 = sc * (NBUF * CHUNK)

            @pl.when(sc > 0)
            def _():
                for b in range(NBUF):
                    pltpu.make_async_copy(
                        out_hbm.at[pl.ds(0, CHUNK)], bufs.at[b], ssem
                    ).wait()

            gathers = []
            for b in range(NBUF):
                gathers.append(pltpu.async_copy(
                    table_hbm.at[idx_all.at[sc * NBUF + b]], bufs.at[b], gsem))
            for b in range(NBUF):
                gathers[b].wait()
                pltpu.async_copy(
                    bufs.at[b],
                    out_hbm.at[pl.ds(base + off0 + b * CHUNK, CHUNK)],
                    ssem)
            return carry

        lax.fori_loop(0, n_super, body, 0)
        for b in range(NBUF):
            pltpu.make_async_copy(
                out_hbm.at[pl.ds(0, CHUNK)], bufs.at[b], ssem).wait()

    return sc_expand


@jax.jit
def kernel(ss_matrix, embed_table, ln_gamma, ln_beta, W):
    b, n, _ = ss_matrix.shape
    total = b * n * n
    table4 = pl.pallas_call(
        _table_block,
        out_shape=jax.ShapeDtypeStruct((N_CLASSES, C_Z), jnp.float32),
    )(embed_table, ln_gamma.reshape(1, C_Z), ln_beta.reshape(1, C_Z), W)
    rows_per_w = total // NW
    ids = ss_matrix.reshape(NW, rows_per_w // CHUNK, CHUNK)
    out = _make_sc_expand(total)(table4, ids)
    return out.reshape(b, n, n, C_Z)


# speedup vs baseline: 22.9303x; 22.9213x over previous
import functools

import jax
import jax.numpy as jnp
from jax import lax
from jax.experimental import pallas as pl
from jax.experimental.pallas import tpu as pltpu, tpu_sc as plsc

N_CLASSES = 4
C_Z = 128
EPS = 1e-5

NC, NS = 2, 16
NW = NC * NS
CHUNK = 128
NBUF = 4
REP = 1024
L = 16


def _table_rep_block(embed_ref, gamma_ref, beta_ref, w_ref, out_ref):
    e = embed_ref[:, :]
    mean = jnp.mean(e, axis=1, keepdims=True)
    var = jnp.mean(jnp.square(e - mean), axis=1, keepdims=True)
    norm = (e - mean) * lax.rsqrt(var + EPS) * gamma_ref[:, :] + beta_ref[:, :]
    table4 = lax.dot_general(norm, w_ref[:, :], (((1,), (1,)), ((), ())),
                             preferred_element_type=jnp.float32)
    out_ref[:, :, :] = jnp.broadcast_to(table4[None, :, :],
                                        (REP, N_CLASSES, C_Z))


def _make_sc_expand(total):
    rows_per_w = total // NW
    n_chunks = rows_per_w // CHUNK
    n_super = n_chunks // NBUF
    mesh = plsc.VectorSubcoreMesh(core_axis_name="c", subcore_axis_name="s")

    @functools.partial(
        pl.kernel,
        mesh=mesh,
        out_type=jax.ShapeDtypeStruct((total, C_Z), jnp.float32),
        scratch_types=[
            pltpu.VMEM((n_chunks, CHUNK), jnp.int32),
            pltpu.VMEM((NBUF, CHUNK, C_Z), jnp.float32),
            pltpu.SemaphoreType.DMA,
            pltpu.SemaphoreType.DMA,
        ],
    )
    def sc_expand(table_hbm, ids_hbm, out_hbm, idx_all, bufs, gsem, ssem):
        wid = lax.axis_index("s") * NC + lax.axis_index("c")
        base = wid * rows_per_w
        pltpu.sync_copy(ids_hbm.at[wid], idx_all)

        lanes = lax.iota(jnp.int32, L) * N_CLASSES

        def spread(v, carry):
            row = v // (CHUNK // L)
            col = (v % (CHUNK // L)) * L
            local = row * CHUNK + col
            off = (local % REP) * N_CLASSES
            sl = (row, pl.ds(col, L))
            idx_all[sl] = idx_all[sl] + lanes + off
            return carry

        lax.fori_loop(0, n_chunks * (CHUNK // L), spread, 0)

        def body(sc, carry):
            off0 = sc * (NBUF * CHUNK)

            @pl.when(sc > 0)
            def _():
                for b in range(NBUF):
                    pltpu.make_async_copy(
                        out_hbm.at[pl.ds(0, CHUNK)], bufs.at[b], ssem
                    ).wait()

            gathers = []
            for b in range(NBUF):
                gathers.append(pltpu.async_copy(
                    table_hbm.at[idx_all.at[sc * NBUF + b]], bufs.at[b], gsem))
            for b in range(NBUF):
                gathers[b].wait()
                pltpu.async_copy(
                    bufs.at[b],
                    out_hbm.at[pl.ds(base + off0 + b * CHUNK, CHUNK)],
                    ssem)
            return carry

        lax.fori_loop(0, n_super, body, 0)
        for b in range(NBUF):
            pltpu.make_async_copy(
                out_hbm.at[pl.ds(0, CHUNK)], bufs.at[b], ssem).wait()

    return sc_expand


@jax.jit
def kernel(ss_matrix, embed_table, ln_gamma, ln_beta, W):
    b, n, _ = ss_matrix.shape
    total = b * n * n
    table_rep = pl.pallas_call(
        _table_rep_block,
        out_shape=jax.ShapeDtypeStruct((REP, N_CLASSES, C_Z), jnp.float32),
    )(embed_table, ln_gamma.reshape(1, C_Z), ln_beta.reshape(1, C_Z), W)
    rows_per_w = total // NW
    ids = ss_matrix.reshape(NW, rows_per_w // CHUNK, CHUNK)
    out = _make_sc_expand(total)(table_rep.reshape(REP * N_CLASSES, C_Z), ids)
    return out.reshape(b, n, n, C_Z)


# speedup vs baseline: 23.5256x; 1.0260x over previous
import functools

import jax
import jax.numpy as jnp
from jax import lax
from jax.experimental import pallas as pl
from jax.experimental.pallas import tpu as pltpu, tpu_sc as plsc

N_CLASSES = 4
C_Z = 128
EPS = 1e-5

NC, NS = 2, 16
NW = NC * NS
CHUNK = 128
NBUF = 4
REP = 4096
L = 16


def _table_rep_block(embed_ref, gamma_ref, beta_ref, w_ref, out_ref):
    e = embed_ref[:, :]
    mean = jnp.mean(e, axis=1, keepdims=True)
    var = jnp.mean(jnp.square(e - mean), axis=1, keepdims=True)
    norm = (e - mean) * lax.rsqrt(var + EPS) * gamma_ref[:, :] + beta_ref[:, :]
    table4 = lax.dot_general(norm, w_ref[:, :], (((1,), (1,)), ((), ())),
                             preferred_element_type=jnp.float32)
    out_ref[:, :, :] = jnp.broadcast_to(table4[None, :, :],
                                        (REP, N_CLASSES, C_Z))


def _make_sc_expand(total):
    rows_per_w = total // NW
    n_chunks = rows_per_w // CHUNK
    n_super = n_chunks // NBUF
    mesh = plsc.VectorSubcoreMesh(core_axis_name="c", subcore_axis_name="s")

    @functools.partial(
        pl.kernel,
        mesh=mesh,
        out_type=jax.ShapeDtypeStruct((total, C_Z), jnp.float32),
        scratch_types=[
            pltpu.VMEM((n_chunks, CHUNK), jnp.int32),
            pltpu.VMEM((NBUF, CHUNK, C_Z), jnp.float32),
            pltpu.SemaphoreType.DMA,
            pltpu.SemaphoreType.DMA,
        ],
    )
    def sc_expand(table_hbm, ids_hbm, out_hbm, idx_all, bufs, gsem, ssem):
        wid = lax.axis_index("s") * NC + lax.axis_index("c")
        base = wid * rows_per_w
        pltpu.sync_copy(ids_hbm.at[wid], idx_all)

        lanes = lax.iota(jnp.int32, L) * N_CLASSES

        def spread(v, carry):
            row = v // (CHUNK // L)
            col = (v % (CHUNK // L)) * L
            local = row * CHUNK + col
            off = (local % REP) * N_CLASSES
            sl = (row, pl.ds(col, L))
            idx_all[sl] = idx_all[sl] + lanes + off
            return carry

        lax.fori_loop(0, n_chunks * (CHUNK // L), spread, 0)

        def body(sc, carry):
            off0 = sc * (NBUF * CHUNK)

            @pl.when(sc > 0)
            def _():
                for b in range(NBUF):
                    pltpu.make_async_copy(
                        out_hbm.at[pl.ds(0, CHUNK)], bufs.at[b], ssem
                    ).wait()

            gathers = []
            for b in range(NBUF):
                gathers.append(pltpu.async_copy(
                    table_hbm.at[idx_all.at[sc * NBUF + b]], bufs.at[b], gsem))
            for b in range(NBUF):
                gathers[b].wait()
                pltpu.async_copy(
                    bufs.at[b],
                    out_hbm.at[pl.ds(base + off0 + b * CHUNK, CHUNK)],
                    ssem)
            return carry

        lax.fori_loop(0, n_super, body, 0)
        for b in range(NBUF):
            pltpu.make_async_copy(
                out_hbm.at[pl.ds(0, CHUNK)], bufs.at[b], ssem).wait()

    return sc_expand


@jax.jit
def kernel(ss_matrix, embed_table, ln_gamma, ln_beta, W):
    b, n, _ = ss_matrix.shape
    total = b * n * n
    table_rep = pl.pallas_call(
        _table_rep_block,
        out_shape=jax.ShapeDtypeStruct((REP, N_CLASSES, C_Z), jnp.float32),
    )(embed_table, ln_gamma.reshape(1, C_Z), ln_beta.reshape(1, C_Z), W)
    rows_per_w = total // NW
    ids = ss_matrix.reshape(NW, rows_per_w // CHUNK, CHUNK)
    out = _make_sc_expand(total)(table_rep.reshape(REP * N_CLASSES, C_Z), ids)
    return out.reshape(b, n, n, C_Z)


# speedup vs baseline: 24.5446x; 1.0433x over previous
import functools

import jax
import jax.numpy as jnp
from jax import lax
from jax.experimental import pallas as pl
from jax.experimental.pallas import tpu as pltpu, tpu_sc as plsc

N_CLASSES = 4
C_Z = 128
EPS = 1e-5

NC, NS = 2, 16
NW = NC * NS
CHUNK = 128
NBUF = 4
REP = 4096
L = 16


def _table_rep_block(embed_ref, gamma_ref, beta_ref, w_ref, out_ref):
    e = embed_ref[:, :]
    mean = jnp.mean(e, axis=1, keepdims=True)
    var = jnp.mean(jnp.square(e - mean), axis=1, keepdims=True)
    norm = (e - mean) * lax.rsqrt(var + EPS) * gamma_ref[:, :] + beta_ref[:, :]
    table4 = lax.dot_general(norm, w_ref[:, :], (((1,), (1,)), ((), ())),
                             preferred_element_type=jnp.float32)
    out_ref[:, :, :] = jnp.broadcast_to(table4[None, :, :],
                                        (REP, N_CLASSES, C_Z))


def _make_sc_expand(total):
    rows_per_w = total // NW
    n_chunks = rows_per_w // CHUNK
    n_super = n_chunks // NBUF
    mesh = plsc.VectorSubcoreMesh(core_axis_name="c", subcore_axis_name="s")

    @functools.partial(
        pl.kernel,
        mesh=mesh,
        out_type=jax.ShapeDtypeStruct((total, C_Z), jnp.float32),
        scratch_types=[
            pltpu.VMEM((n_chunks, CHUNK), jnp.int32),
            pltpu.VMEM((NBUF, CHUNK, C_Z), jnp.float32),
            pltpu.SemaphoreType.DMA,
            pltpu.SemaphoreType.DMA,
        ],
    )
    def sc_expand(table_hbm, ids_hbm, out_hbm, idx_all, bufs, gsem, ssem):
        wid = lax.axis_index("s") * NC + lax.axis_index("c")
        base = wid * rows_per_w
        pltpu.sync_copy(ids_hbm.at[wid], idx_all)

        lanes = lax.iota(jnp.int32, L) * N_CLASSES
        phase = wid * (REP // NW)

        def spread(v, carry):
            row = v // (CHUNK // L)
            col = (v % (CHUNK // L)) * L
            local = row * CHUNK + col
            off = ((local + phase) % REP) * N_CLASSES
            sl = (row, pl.ds(col, L))
            idx_all[sl] = idx_all[sl] + lanes + off
            return carry

        lax.fori_loop(0, n_chunks * (CHUNK // L), spread, 0)

        def body(sc, carry):
            off0 = sc * (NBUF * CHUNK)

            @pl.when(sc > 0)
            def _():
                for b in range(NBUF):
                    pltpu.make_async_copy(
                        out_hbm.at[pl.ds(0, CHUNK)], bufs.at[b], ssem
                    ).wait()

            gathers = []
            for b in range(NBUF):
                gathers.append(pltpu.async_copy(
                    table_hbm.at[idx_all.at[sc * NBUF + b]], bufs.at[b], gsem))
            for b in range(NBUF):
                gathers[b].wait()
                pltpu.async_copy(
                    bufs.at[b],
                    out_hbm.at[pl.ds(base + off0 + b * CHUNK, CHUNK)],
                    ssem)
            return carry

        lax.fori_loop(0, n_super, body, 0)
        for b in range(NBUF):
            pltpu.make_async_copy(
                out_hbm.at[pl.ds(0, CHUNK)], bufs.at[b], ssem).wait()

    return sc_expand


@jax.jit
def kernel(ss_matrix, embed_table, ln_gamma, ln_beta, W):
    b, n, _ = ss_matrix.shape
    total = b * n * n
    table_rep = pl.pallas_call(
        _table_rep_block,
        out_shape=jax.ShapeDtypeStruct((REP, N_CLASSES, C_Z), jnp.float32),
    )(embed_table, ln_gamma.reshape(1, C_Z), ln_beta.reshape(1, C_Z), W)
    rows_per_w = total // NW
    ids = ss_matrix.reshape(NW, rows_per_w // CHUNK, CHUNK)
    out = _make_sc_expand(total)(table_rep.reshape(REP * N_CLASSES, C_Z), ids)
    return out.reshape(b, n, n, C_Z)
